# Initial kernel scaffold; baseline (speedup 1.0000x reference)
import jax
import jax.numpy as jnp
from jax.experimental import pallas as pl


def kernel(pred1_logits, pred2_logits, label_table):
    raise NotImplementedError("write your pallas kernel here")


# speedup vs baseline: 2.0837x; 2.0837x over previous
import functools

import jax
import jax.numpy as jnp
from jax.experimental import pallas as pl

C1 = 10
C2 = 1000
BATCH = 16384
BB = 1024


def _loss_body(p1_ref, x_ref, tab_ref, out_ref):
    x = x_ref[...]
    m = jnp.max(x, axis=1, keepdims=True)
    lse = m[:, 0] + jnp.log(jnp.sum(jnp.exp(x - m), axis=1))

    p1 = p1_ref[...]
    col = jax.lax.broadcasted_iota(jnp.int32, (BB, C1), 1)
    pm = jnp.max(p1, axis=1, keepdims=True)
    first_idx = jnp.min(jnp.where(p1 == pm, col, C1), axis=1, keepdims=True)
    onehot = (col == first_idx).astype(jnp.float32)
    labels = jnp.dot(onehot, tab_ref[...], preferred_element_type=jnp.float32)
    t = jnp.sum(labels * x, axis=1)

    partial = (jnp.sum(lse - t) * (1.0 / BATCH)).reshape(1, 1)

    @pl.when(pl.program_id(0) == 0)
    def _():
        out_ref[...] = jnp.zeros((1, 1), jnp.float32)

    out_ref[...] += partial


@functools.partial(jax.jit, static_argnames=())
def kernel(pred1_logits, pred2_logits, label_table):
    grid = BATCH // BB
    out = pl.pallas_call(
        _loss_body,
        grid=(grid,),
        in_specs=[
            pl.BlockSpec((BB, C1), lambda i: (i, 0)),
            pl.BlockSpec((BB, C2), lambda i: (i, 0)),
            pl.BlockSpec((C1, C2), lambda i: (0, 0)),
        ],
        out_specs=pl.BlockSpec((1, 1), lambda i: (0, 0)),
        out_shape=jax.ShapeDtypeStruct((1, 1), jnp.float32),
    )(pred1_logits, pred2_logits, label_table)
    return out[0, 0]


# speedup vs baseline: 2.1981x; 1.0549x over previous
import functools

import jax
import jax.numpy as jnp
from jax.experimental import pallas as pl

C1 = 10
C2 = 1000
BATCH = 16384
BB = 4096


def _loss_body(p1_ref, x_ref, tab_ref, out_ref):
    x = x_ref[...]
    m = jnp.max(x, axis=1, keepdims=True)
    lse = m[:, 0] + jnp.log(jnp.sum(jnp.exp(x - m), axis=1))

    p1 = p1_ref[...]
    col = jax.lax.broadcasted_iota(jnp.int32, (BB, C1), 1)
    pm = jnp.max(p1, axis=1, keepdims=True)
    first_idx = jnp.min(jnp.where(p1 == pm, col, C1), axis=1, keepdims=True)
    onehot = (col == first_idx).astype(jnp.float32)
    labels = jnp.dot(onehot, tab_ref[...], preferred_element_type=jnp.float32)
    t = jnp.sum(labels * x, axis=1)

    partial = (jnp.sum(lse - t) * (1.0 / BATCH)).reshape(1, 1)

    @pl.when(pl.program_id(0) == 0)
    def _():
        out_ref[...] = jnp.zeros((1, 1), jnp.float32)

    out_ref[...] += partial


@jax.jit
def kernel(pred1_logits, pred2_logits, label_table):
    grid = BATCH // BB
    out = pl.pallas_call(
        _loss_body,
        grid=(grid,),
        in_specs=[
            pl.BlockSpec((BB, C1), lambda i: (i, 0)),
            pl.BlockSpec((BB, C2), lambda i: (i, 0)),
            pl.BlockSpec((C1, C2), lambda i: (0, 0)),
        ],
        out_specs=pl.BlockSpec((1, 1), lambda i: (0, 0)),
        out_shape=jax.ShapeDtypeStruct((1, 1), jnp.float32),
    )(pred1_logits, pred2_logits, label_table)
    return out[0, 0]
